# Initial kernel scaffold; baseline (speedup 1.0000x reference)
import functools
import jax
import jax.numpy as jnp
from jax import lax
from jax.experimental import pallas as pl
from jax.experimental.pallas import tpu as pltpu
from jax.experimental.pallas import tpu_sc as plsc

N = 100000
E = 6400000
NC = 2
NS = 16
NW = NC * NS
L = 16
CHUNK = 2048
ROWS = CHUNK // 128
NCH = E // CHUNK
CPW = (NCH + NW - 1) // NW
NPT = 6272
NPAD = NS * NPT

_mesh = plsc.VectorSubcoreMesh(
    core_axis_name="c", subcore_axis_name="s", num_cores=NC)


@functools.partial(
    pl.kernel,
    out_type=jax.ShapeDtypeStruct((NC, NPAD), jnp.float32),
    mesh=_mesh,
    scratch_types=[
        pltpu.VMEM((N,), jnp.float32),
        pltpu.VMEM((CHUNK,), jnp.int32),
        pltpu.VMEM((ROWS, 128), jnp.int32),
        pltpu.VMEM((CHUNK,), jnp.float32),
        pltpu.VMEM((CHUNK,), jnp.float32),
        pltpu.VMEM((NPT,), jnp.float32),
        pltpu.VMEM_SHARED((NPAD,), jnp.float32),
        pltpu.SemaphoreType.DMA,
    ],
)
def _sc_scatter(ei4_hbm, eif_hbm, w_hbm, vals_hbm, out_hbm,
                vals_v, src_v, dst_v, w_v, msg_v, zbuf, acc_sh, sem):
    cid = lax.axis_index("c")
    sid = lax.axis_index("s")
    wid = sid * NC + cid

    zeros = jnp.zeros((L,), jnp.float32)

    def _z(i, carry):
        zbuf[pl.ds(i * L, L)] = zeros
        return carry

    lax.fori_loop(0, NPT // L, _z, 0)
    pltpu.sync_copy(zbuf, acc_sh.at[pl.ds(sid * NPT, NPT)])

    pltpu.sync_copy(vals_hbm, vals_v)

    plsc.subcore_barrier()

    def _chunk(i, carry):
        c = i * NW + wid

        @pl.when(c < NCH)
        def _():
            base = pl.multiple_of(c * CHUNK, 8)
            pltpu.sync_copy(eif_hbm.at[0, pl.ds(base, CHUNK)], src_v)
            pltpu.sync_copy(ei4_hbm.at[1, c], dst_v)
            pltpu.sync_copy(w_hbm.at[pl.ds(base, CHUNK)], w_v)

            def _g(j, acc):
                o = j * L
                idx = src_v[pl.ds(o, L)]
                v = plsc.load_gather(vals_v, [idx])
                msg_v[pl.ds(o, L)] = v * w_v[pl.ds(o, L)]
                return acc

            lax.fori_loop(0, CHUNK // L, _g, 0)

            def _s(k, acc):
                pltpu.sync_copy(msg_v.at[pl.ds(k * 128, 128)],
                                acc_sh.at[dst_v.at[k]], add=True)
                return acc

            lax.fori_loop(0, ROWS, _s, 0)

        return carry

    lax.fori_loop(0, CPW, _chunk, 0)

    plsc.subcore_barrier()
    pltpu.sync_copy(acc_sh.at[pl.ds(sid * NPT, NPT)],
                    out_hbm.at[cid, pl.ds(sid * NPT, NPT)])


def _combine_body(x_ref, o_ref):
    o_ref[...] = jax.nn.sigmoid(x_ref[0] + x_ref[1])


_combine = pl.pallas_call(
    _combine_body,
    out_shape=jax.ShapeDtypeStruct((NPAD // 128, 128), jnp.float32),
)


@jax.jit
def kernel(values, edge_index, edge_weight):
    ei4 = edge_index.reshape(2, NCH, ROWS, 128)
    partials = _sc_scatter(ei4, edge_index, edge_weight, values)
    out = _combine(partials.reshape(NC, NPAD // 128, 128))
    return out.reshape(NPAD)[:N]


# speedup vs baseline: 138.0579x; 138.0579x over previous
import functools
import jax
import jax.numpy as jnp
from jax import lax
from jax.experimental import pallas as pl
from jax.experimental.pallas import tpu as pltpu
from jax.experimental.pallas import tpu_sc as plsc

N = 100000
E = 6400000
NC = 2
NS = 16
NW = NC * NS
L = 16
CHUNK = 2048
ROWS = CHUNK // 128
NCH = E // CHUNK
CPW = (NCH + NW - 1) // NW
NPT = 6272
NPAD = NS * NPT

_mesh = plsc.VectorSubcoreMesh(
    core_axis_name="c", subcore_axis_name="s", num_cores=NC)


@functools.partial(
    pl.kernel,
    out_type=jax.ShapeDtypeStruct((NC, NPAD), jnp.float32),
    mesh=_mesh,
    scratch_types=[
        pltpu.VMEM((N,), jnp.float32),
        pltpu.VMEM((CHUNK,), jnp.int32),
        pltpu.VMEM((ROWS, 128), jnp.int32),
        pltpu.VMEM((CHUNK,), jnp.float32),
        pltpu.VMEM((CHUNK,), jnp.float32),
        pltpu.VMEM((NPT,), jnp.float32),
        pltpu.VMEM_SHARED((NPAD,), jnp.float32),
        pltpu.SemaphoreType.DMA,
    ],
    compiler_params=pltpu.CompilerParams(needs_layout_passes=False),
)
def _sc_scatter(ei4_hbm, eif_hbm, w_hbm, vals_hbm, out_hbm,
                vals_v, src_v, dst_v, w_v, msg_v, zbuf, acc_sh, sem):
    cid = lax.axis_index("c")
    sid = lax.axis_index("s")
    wid = sid * NC + cid

    zeros = jnp.zeros((L,), jnp.float32)

    def _z(i, carry):
        zbuf[pl.ds(i * L, L)] = zeros
        return carry

    lax.fori_loop(0, NPT // L, _z, 0)
    pltpu.sync_copy(zbuf, acc_sh.at[pl.ds(sid * NPT, NPT)])

    pltpu.sync_copy(vals_hbm, vals_v)

    plsc.subcore_barrier()

    def _chunk(i, carry):
        c = i * NW + wid

        @pl.when(c < NCH)
        def _():
            base = pl.multiple_of(c * CHUNK, 8)
            pltpu.sync_copy(eif_hbm.at[0, pl.ds(base, CHUNK)], src_v)
            pltpu.sync_copy(ei4_hbm.at[1, c], dst_v)
            pltpu.sync_copy(w_hbm.at[pl.ds(base, CHUNK)], w_v)

            def _g(j, acc):
                o = j * L
                idx = src_v[pl.ds(o, L)]
                v = plsc.load_gather(vals_v, [idx])
                msg_v[pl.ds(o, L)] = v * w_v[pl.ds(o, L)]
                return acc

            lax.fori_loop(0, CHUNK // L, _g, 0)

            def _s(k, acc):
                pltpu.sync_copy(msg_v.at[pl.ds(k * 128, 128)],
                                acc_sh.at[dst_v.at[k]], add=True)
                return acc

            lax.fori_loop(0, ROWS, _s, 0)

        return carry

    lax.fori_loop(0, CPW, _chunk, 0)

    plsc.subcore_barrier()
    pltpu.sync_copy(acc_sh.at[pl.ds(sid * NPT, NPT)],
                    out_hbm.at[cid, pl.ds(sid * NPT, NPT)])


def _combine_body(x_ref, o_ref):
    o_ref[...] = jax.nn.sigmoid(x_ref[0] + x_ref[1])


_combine = pl.pallas_call(
    _combine_body,
    out_shape=jax.ShapeDtypeStruct((NPAD // 128, 128), jnp.float32),
)


@jax.jit
def kernel(values, edge_index, edge_weight):
    ei4 = edge_index.reshape(2, NCH, ROWS, 128)
    partials = _sc_scatter(ei4, edge_index, edge_weight, values)
    out = _combine(partials.reshape(NC, NPAD // 128, 128))
    return out.reshape(NPAD)[:N]


# speedup vs baseline: 170.8231x; 1.2373x over previous
import functools
import jax
import jax.numpy as jnp
from jax import lax
from jax.experimental import pallas as pl
from jax.experimental.pallas import tpu as pltpu
from jax.experimental.pallas import tpu_sc as plsc

N = 100000
E = 6400000
NC = 2
NS = 16
NW = NC * NS
L = 16
CHUNK = 2048
ROWS = CHUNK // 128
NCH = E // CHUNK
CPW = (NCH + NW - 1) // NW
NPT = 6272
NPAD = NS * NPT

_mesh = plsc.VectorSubcoreMesh(
    core_axis_name="c", subcore_axis_name="s", num_cores=NC)


@functools.partial(
    pl.kernel,
    out_type=jax.ShapeDtypeStruct((NC, NPAD), jnp.float32),
    mesh=_mesh,
    scratch_types=[
        pltpu.VMEM((N,), jnp.float32),
        pltpu.VMEM((CHUNK,), jnp.int32),
        pltpu.VMEM((CHUNK,), jnp.int32),
        pltpu.VMEM((CHUNK,), jnp.float32),
        pltpu.VMEM((CHUNK,), jnp.float32),
        pltpu.VMEM((NPT,), jnp.float32),
        pltpu.VMEM_SHARED((NPAD,), jnp.float32),
        pltpu.SemaphoreType.DMA,
    ],
    compiler_params=pltpu.CompilerParams(needs_layout_passes=False),
)
def _sc_scatter(ei4_hbm, eif_hbm, w_hbm, vals_hbm, out_hbm,
                vals_v, src_v, dst_v, w_v, msg_v, zbuf, acc_sh, sem):
    cid = lax.axis_index("c")
    sid = lax.axis_index("s")
    wid = sid * NC + cid

    zeros = jnp.zeros((L,), jnp.float32)

    def _z(i, carry):
        zbuf[pl.ds(i * L, L)] = zeros
        return carry

    lax.fori_loop(0, NPT // L, _z, 0)
    pltpu.sync_copy(zbuf, acc_sh.at[pl.ds(sid * NPT, NPT)])

    pltpu.sync_copy(vals_hbm, vals_v)

    plsc.subcore_barrier()

    def _chunk(i, carry):
        c = i * NW + wid

        @pl.when(c < NCH)
        def _():
            base = pl.multiple_of(c * CHUNK, 8)
            pltpu.sync_copy(eif_hbm.at[0, pl.ds(base, CHUNK)], src_v)
            pltpu.sync_copy(eif_hbm.at[1, pl.ds(base, CHUNK)], dst_v)
            pltpu.sync_copy(w_hbm.at[pl.ds(base, CHUNK)], w_v)

            def _g(j, acc):
                o = j * L
                idx = src_v[pl.ds(o, L)]
                v = plsc.load_gather(vals_v, [idx])
                msg_v[pl.ds(o, L)] = v * w_v[pl.ds(o, L)]
                return acc

            lax.fori_loop(0, CHUNK // L, _g, 0)

            pltpu.sync_copy(msg_v, acc_sh.at[dst_v], add=True)

        return carry

    lax.fori_loop(0, CPW, _chunk, 0)

    plsc.subcore_barrier()
    pltpu.sync_copy(acc_sh.at[pl.ds(sid * NPT, NPT)],
                    out_hbm.at[cid, pl.ds(sid * NPT, NPT)])


def _combine_body(x_ref, o_ref):
    o_ref[...] = jax.nn.sigmoid(x_ref[0] + x_ref[1])


_combine = pl.pallas_call(
    _combine_body,
    out_shape=jax.ShapeDtypeStruct((NPAD // 128, 128), jnp.float32),
)


@jax.jit
def kernel(values, edge_index, edge_weight):
    ei4 = edge_index.reshape(2, NCH, ROWS, 128)
    partials = _sc_scatter(ei4, edge_index, edge_weight, values)
    out = _combine(partials.reshape(NC, NPAD // 128, 128))
    return out.reshape(NPAD)[:N]


# speedup vs baseline: 584.0761x; 3.4192x over previous
import functools
import jax
import jax.numpy as jnp
from jax import lax
from jax.experimental import pallas as pl
from jax.experimental.pallas import tpu as pltpu
from jax.experimental.pallas import tpu_sc as plsc

N = 100000
E = 6400000
NC = 2
NS = 16
NW = NC * NS
L = 16
CHUNK = 2048
NCH = E // CHUNK
SUBIT = (NCH + NW - 1) // NW + 1
MACRO = SUBIT // 3
NPT = 6272
NPAD = NS * NPT

_mesh = plsc.VectorSubcoreMesh(
    core_axis_name="c", subcore_axis_name="s", num_cores=NC)


@functools.partial(
    pl.kernel,
    out_type=jax.ShapeDtypeStruct((NC, NPAD), jnp.float32),
    mesh=_mesh,
    scratch_types=[
        pltpu.VMEM((N,), jnp.float32),
        pltpu.VMEM((CHUNK,), jnp.int32),
        pltpu.VMEM((CHUNK,), jnp.int32),
        pltpu.VMEM((CHUNK,), jnp.int32),
        pltpu.VMEM((CHUNK,), jnp.int32),
        pltpu.VMEM((CHUNK,), jnp.int32),
        pltpu.VMEM((CHUNK,), jnp.int32),
        pltpu.VMEM((CHUNK,), jnp.float32),
        pltpu.VMEM((CHUNK,), jnp.float32),
        pltpu.VMEM((CHUNK,), jnp.float32),
        pltpu.VMEM((CHUNK,), jnp.float32),
        pltpu.VMEM((CHUNK,), jnp.float32),
        pltpu.VMEM((CHUNK,), jnp.float32),
        pltpu.VMEM_SHARED((NPAD,), jnp.float32),
        pltpu.SemaphoreType.DMA((3,)),
        pltpu.SemaphoreType.DMA((3,)),
    ],
    compiler_params=pltpu.CompilerParams(needs_layout_passes=False),
)
def _sc_scatter(eif_hbm, w_hbm, vals_hbm, out_hbm, vals_v,
                src_a, src_b, src_c, dst_a, dst_b, dst_c,
                w_a, w_b, w_c, msg_a, msg_b, msg_c,
                acc_sh, sem_in, sem_sc):
    srcs = (src_a, src_b, src_c)
    dsts = (dst_a, dst_b, dst_c)
    ws = (w_a, w_b, w_c)
    msgs = (msg_a, msg_b, msg_c)
    cid = lax.axis_index("c")
    sid = lax.axis_index("s")
    wid = sid * NC + cid

    def _fire_in(c, j):
        base = pl.multiple_of(c * CHUNK, 8)
        pltpu.async_copy(eif_hbm.at[0, pl.ds(base, CHUNK)], srcs[j],
                         sem_in.at[j])
        pltpu.async_copy(eif_hbm.at[1, pl.ds(base, CHUNK)], dsts[j],
                         sem_in.at[j])
        pltpu.async_copy(w_hbm.at[pl.ds(base, CHUNK)], ws[j],
                         sem_in.at[j])

    def _wait_in(c, j):
        base = pl.multiple_of(c * CHUNK, 8)
        pltpu.make_async_copy(eif_hbm.at[0, pl.ds(base, CHUNK)], srcs[j],
                              sem_in.at[j]).wait()
        pltpu.make_async_copy(eif_hbm.at[1, pl.ds(base, CHUNK)], dsts[j],
                              sem_in.at[j]).wait()
        pltpu.make_async_copy(w_hbm.at[pl.ds(base, CHUNK)], ws[j],
                              sem_in.at[j]).wait()

    def _wait_scatter(j):
        pltpu.make_async_copy(msgs[j], acc_sh.at[dsts[j]],
                              sem_sc.at[j]).wait()

    _fire_in(wid, 0)
    _fire_in(NW + wid, 1)

    zeros = jnp.zeros((L,), jnp.float32)

    def _z(i, carry):
        msg_c[pl.ds(i * L, L)] = zeros
        return carry

    lax.fori_loop(0, CHUNK // L, _z, 0)
    for k in range(NPT // CHUNK):
        pltpu.sync_copy(msg_c,
                        acc_sh.at[pl.ds(sid * NPT + k * CHUNK, CHUNK)])
    pltpu.sync_copy(msg_c.at[pl.ds(0, NPT % CHUNK)],
                    acc_sh.at[pl.ds(sid * NPT + NPT - NPT % CHUNK,
                                    NPT % CHUNK)])

    pltpu.sync_copy(vals_hbm, vals_v)

    plsc.subcore_barrier()

    def _macro(i, carry):
        for j in range(3):
            i3 = i * 3 + j
            c = i3 * NW + wid

            @pl.when(c < NCH)
            def _():
                _wait_in(c, j)

                def _g(g, acc):
                    o = g * L
                    idx = srcs[j][pl.ds(o, L)]
                    v = plsc.load_gather(vals_v, [idx])
                    msgs[j][pl.ds(o, L)] = v * ws[j][pl.ds(o, L)]
                    return acc

                lax.fori_loop(0, CHUNK // L, _g, 0)
                pltpu.async_copy(msgs[j], acc_sh.at[dsts[j]],
                                 sem_sc.at[j], add=True)

            jp = (j + 2) % 3
            cprev = c - NW

            @pl.when((i3 >= 1) & (cprev < NCH))
            def _():
                _wait_scatter(jp)

            cnext = c + 2 * NW

            @pl.when(cnext < NCH)
            def _():
                _fire_in(cnext, jp)

        return carry

    lax.fori_loop(0, MACRO, _macro, 0)

    plsc.subcore_barrier()
    pltpu.sync_copy(acc_sh.at[pl.ds(sid * NPT, NPT)],
                    out_hbm.at[cid, pl.ds(sid * NPT, NPT)])


def _combine_body(x_ref, o_ref):
    o_ref[...] = jax.nn.sigmoid(x_ref[0] + x_ref[1])


_combine = pl.pallas_call(
    _combine_body,
    out_shape=jax.ShapeDtypeStruct((NPAD // 128, 128), jnp.float32),
)


@jax.jit
def kernel(values, edge_index, edge_weight):
    partials = _sc_scatter(edge_index, edge_weight, values)
    out = _combine(partials.reshape(NC, NPAD // 128, 128))
    return out.reshape(NPAD)[:N]
